# baseline (speedup 1.0000x reference)
import jax
import jax.numpy as jnp
import numpy as np
from jax import lax
from jax.experimental import pallas as pl
from jax.experimental.pallas import tpu as pltpu
from jax.experimental.pallas import tpu_sc as plsc

_B, _N, _G, _R = 4, 5000, 50, 64
_NP = 5120
_NEG_OV = 0.3
_POS_OV = 0.7
_NUM_FG = 128
_BATCH = 256
_CHUNK = 512
_PAD_COORD = -1.0e4


def _tf2x32(k1, k2, x1, x2):
    R0 = (13, 15, 26, 6)
    R1 = (17, 29, 16, 24)
    ks = [np.uint32(k1), np.uint32(k2)]
    ks.append(ks[0] ^ ks[1] ^ np.uint32(0x1BD11BDA))
    x = [x1.astype(np.uint32) + ks[0], x2.astype(np.uint32) + ks[1]]

    def rounds(x, rots):
        for r in rots:
            x0 = x[0] + x[1]
            x1r = (x[1] << np.uint32(r)) | (x[1] >> np.uint32(32 - r))
            x = [x0, x0 ^ x1r]
        return x

    for i, (rots, ka, kb) in enumerate(
            [(R0, 1, 2), (R1, 2, 0), (R0, 0, 1), (R1, 1, 2), (R0, 2, 0)]):
        x = rounds(x, rots)
        x = [x[0] + ks[ka], x[1] + ks[kb] + np.uint32(i + 1)]
    return x


def _np_uniform(key, shape):
    n = int(np.prod(shape))
    lo = np.arange(n, dtype=np.uint32)
    hi = np.zeros(n, dtype=np.uint32)
    b1, b2 = _tf2x32(key[0], key[1], hi, lo)
    bits = b1 ^ b2
    fb = (bits >> np.uint32(9)) | np.uint32(0x3F800000)
    return (fb.view(np.float32) - np.float32(1.0)).reshape(shape)


def _np_key42_pris():
    b1, b2 = _tf2x32(np.uint32(0), np.uint32(42),
                     np.zeros(2, np.uint32), np.arange(2, dtype=np.uint32))
    keys = np.stack([b1, b2], axis=1)
    return (_np_uniform(keys[0], (_B, _N)), _np_uniform(keys[1], (_B, _N)))


_PRI1, _PRI2 = _np_key42_pris()


def _pad_bits(pri):
    out = np.full((_B, _NP), -1, np.int32)
    out[:, :_N] = pri.view(np.int32)
    return out


_PB1 = _pad_bits(_PRI1)
_PB2 = _pad_bits(_PRI2)


def _np_order(pri):
    bits = pri.view(np.int32)
    order = np.argsort(-pri, axis=1)
    ordp = np.concatenate(
        [order, np.tile(np.arange(_N, _NP), (_B, 1))], axis=1).astype(np.int32)
    pv = np.take_along_axis(bits, order, axis=1)
    pvp = np.concatenate([pv, np.full((_B, _NP - _N), -1, np.int32)], axis=1)
    return ordp, pvp


_ORD1, _PV1 = _np_order(_PRI1)
_ORD2, _PV2 = _np_order(_PRI2)


def _kth_threshold(mask, bits, k):
    def body(_, lohi):
        lo, hi = lohi
        mid = (lo + hi) // 2
        cnt = jnp.sum((mask & (bits >= mid)).astype(jnp.int32), axis=1,
                      keepdims=True)
        ge = cnt >= k
        return (jnp.where(ge, mid, lo), jnp.where(ge, hi, mid))

    init = (jnp.zeros((_B, 1), jnp.int32),
            jnp.full((_B, 1), 0x40000000, jnp.int32))
    lo, _ = lax.fori_loop(0, 31, body, init)
    return lo


def _body(planes_ref, gtb_ref, oh_s_ref, oh_o_ref, score_ref,
          labels_ref, stats_ref, ov_scr):
    for b in range(_B):
        oh_s = oh_s_ref[b]
        oh_o = oh_o_ref[b]
        gtb = gtb_ref[b]

        def sel(oh, row):
            return jnp.sum(oh * gtb[row:row + 1, :], axis=1, keepdims=True)

        gsx1, gsy1, gsx2, gsy2 = (sel(oh_s, 0), sel(oh_s, 1), sel(oh_s, 2),
                                  sel(oh_s, 3))
        gox1, goy1, gox2, goy2 = (sel(oh_o, 0), sel(oh_o, 1), sel(oh_o, 2),
                                  sel(oh_o, 3))
        vld = jnp.sum(oh_s, axis=1, keepdims=True) > 0.0
        ags = (gsx2 - gsx1 + 1.0) * (gsy2 - gsy1 + 1.0)
        ago = (gox2 - gox1 + 1.0) * (goy2 - goy1 + 1.0)

        for j in range(_NP // _CHUNK):
            ch = planes_ref[b, :, j * _CHUNK:(j + 1) * _CHUNK]
            rsx1, rsy1, rsx2, rsy2 = (ch[0:1], ch[1:2], ch[2:3], ch[3:4])
            rox1, roy1, rox2, roy2 = (ch[4:5], ch[5:6], ch[6:7], ch[7:8])
            ars = (rsx2 - rsx1 + 1.0) * (rsy2 - rsy1 + 1.0)
            aro = (rox2 - rox1 + 1.0) * (roy2 - roy1 + 1.0)

            iw_s = jnp.minimum(rsx2, gsx2) - jnp.maximum(rsx1, gsx1) + 1.0
            ih_s = jnp.minimum(rsy2, gsy2) - jnp.maximum(rsy1, gsy1) + 1.0
            inter_s = jnp.clip(iw_s, 0.0) * jnp.clip(ih_s, 0.0)
            iou_s = inter_s / (ars + ags - inter_s)

            iw_o = jnp.minimum(rox2, gox2) - jnp.maximum(rox1, gox1) + 1.0
            ih_o = jnp.minimum(roy2, goy2) - jnp.maximum(roy1, goy1) + 1.0
            inter_o = jnp.clip(iw_o, 0.0) * jnp.clip(ih_o, 0.0)
            iou_o = inter_o / (aro + ago - inter_o)

            ov_scr[b * _R:(b + 1) * _R, j * _CHUNK:(j + 1) * _CHUNK] = (
                jnp.where(vld, iou_s * iou_o, 0.0))

    ov = ov_scr[:, :]
    gmax = jnp.max(ov, axis=1, keepdims=True)
    gmax_adj = jnp.where(gmax == 0.0, 1e-5, gmax)
    maxov_rows = []
    keep_rows = []
    for b in range(_B):
        ovb = ov[b * _R:(b + 1) * _R]
        maxov_rows.append(jnp.max(ovb, axis=0, keepdims=True))
        keep_rows.append(
            jnp.any(ovb == gmax_adj[b * _R:(b + 1) * _R], axis=0,
                    keepdims=True).astype(jnp.float32))
    maxov = jnp.concatenate(maxov_rows, axis=0)
    keep = jnp.concatenate(keep_rows, axis=0) > 0.0

    labels = jnp.full((_B, _NP), -1.0, jnp.float32)
    labels = jnp.where(maxov < _NEG_OV, 0.0, labels)
    labels = jnp.where(keep, 1.0, labels)
    labels = jnp.where(maxov >= _POS_OV, 1.0, labels)
    idx = lax.broadcasted_iota(jnp.int32, (_B, _NP), 1)
    labels = jnp.where(idx < _N, labels, -1.0)

    score = score_ref[:, :]
    fg = labels == 1.0
    bg = labels == 0.0
    pre = (score > 0.8) & (idx < 300)
    all_fg = jnp.sum(fg.astype(jnp.float32))
    all_bg = jnp.sum(bg.astype(jnp.float32))
    rights_fg = jnp.sum((fg & (score >= 0.5)).astype(jnp.float32))
    rights_bg = jnp.sum((bg & (score < 0.5)).astype(jnp.float32))
    n_pre_correct = jnp.sum((pre & fg).astype(jnp.float32))
    all_fg_pre = jnp.sum(pre.astype(jnp.float32))
    r_fg = rights_fg / jnp.maximum(all_fg, 1.0)
    r_bg = rights_bg / jnp.maximum(all_bg, 1.0)
    acc = n_pre_correct / (1e-5 + all_fg_pre)
    row = lax.broadcasted_iota(jnp.int32, (8, 128), 0)
    stats_ref[...] = jnp.where(
        row == 0, r_fg, jnp.where(row == 1, r_bg, jnp.where(row == 2, acc,
                                                            0.0)))

    labels_ref[...] = labels


_SC_VECS = _NP // 16
_INT_MIN = -2147483647 - 1


def _sc_body(lab_hbm, pb1_hbm, pb2_hbm, ord1_hbm, pv1_hbm, ord2_hbm, pv2_hbm,
             out_hbm, lab_v, pb_v, ord_v, pv_v):
    c = lax.axis_index("c")
    s = lax.axis_index("s")
    b = c * 2 + s

    @pl.when(s < 2)
    def _():
        pltpu.sync_copy(lab_hbm.at[b], lab_v)

        def count_body(i, acc):
            fa, ba = acc
            lv = lab_v[pl.ds(i * 16, 16)]
            return (fa + (lv == 1.0).astype(jnp.int32),
                    ba + (lv == 0.0).astype(jnp.int32))

        z = jnp.zeros((16,), jnp.int32)
        fa, ba = lax.fori_loop(0, _SC_VECS, count_body, (z, z))
        fg_cnt = jnp.sum(fa)
        bg_cnt = jnp.sum(ba)

        def walk(tgt, k):
            def cond(st):
                i, cnt, _ = st
                return (i < _SC_VECS) & (cnt < k)

            def body(st):
                i, cnt, thr = st
                ov = ord_v[pl.ds(i * 16, 16)]
                lg = plsc.load_gather(lab_v, [ov])
                mi = (lg == tgt).astype(jnp.int32)
                cs = plsc.cumsum(mi)
                tgt_k = jnp.full((16,), k - cnt, jnp.int32)
                hit = (cs == tgt_k) & (mi > 0)
                pvv = pv_v[pl.ds(i * 16, 16)]
                hitv = jnp.max(jnp.where(hit, pvv, jnp.zeros((16,), jnp.int32)))
                return (i + 1, cnt + jnp.sum(mi), jnp.maximum(thr, hitv))

            _, _, thr = lax.while_loop(
                cond, body, (jnp.int32(0), jnp.int32(0), jnp.int32(0)))
            return thr

        pltpu.sync_copy(ord1_hbm.at[b], ord_v)
        pltpu.sync_copy(pv1_hbm.at[b], pv_v)
        pltpu.sync_copy(pb1_hbm.at[b], pb_v)
        thr1 = walk(1.0, jnp.int32(_NUM_FG))
        gate1 = fg_cnt > _NUM_FG
        thr1e = jnp.where(gate1, thr1, jnp.int32(_INT_MIN))

        thr1ev = jnp.full((16,), thr1e, jnp.int32)
        thr1v = jnp.full((16,), thr1, jnp.int32)

        def fg_apply(i, sel):
            lv = lab_v[pl.ds(i * 16, 16)]
            pb = pb_v[pl.ds(i * 16, 16)]
            fgm = lv == 1.0
            lab_v[pl.ds(i * 16, 16)] = jnp.where(fgm & (pb < thr1ev), -1.0, lv)
            return sel + (fgm & (pb >= thr1v)).astype(jnp.int32)

        sel = lax.fori_loop(0, _SC_VECS, fg_apply, z)
        fg_sel = jnp.where(gate1, jnp.sum(sel), fg_cnt)
        num_bg = _BATCH - fg_sel

        pltpu.sync_copy(ord2_hbm.at[b], ord_v)
        pltpu.sync_copy(pv2_hbm.at[b], pv_v)
        pltpu.sync_copy(pb2_hbm.at[b], pb_v)
        thr2 = walk(0.0, jnp.clip(num_bg, 1, _N))
        gate2 = bg_cnt > num_bg
        thr2e = jnp.where(gate2, thr2, jnp.int32(_INT_MIN))

        thr2ev = jnp.full((16,), thr2e, jnp.int32)

        def bg_apply(i, carry):
            lv = lab_v[pl.ds(i * 16, 16)]
            pb = pb_v[pl.ds(i * 16, 16)]
            lab_v[pl.ds(i * 16, 16)] = jnp.where(
                (lv == 0.0) & (pb < thr2ev), -1.0, lv)
            return carry

        lax.fori_loop(0, _SC_VECS, bg_apply, jnp.int32(0))
        pltpu.sync_copy(lab_v, out_hbm.at[b])


@jax.jit
def _run(planes, gtb, oh_s, oh_o, score, pb1, pb2, ord1, pv1, ord2, pv2):
    labels_pre, stats = pl.pallas_call(
        _body,
        out_shape=[
            jax.ShapeDtypeStruct((_B, _NP), jnp.float32),
            jax.ShapeDtypeStruct((8, 128), jnp.float32),
        ],
        scratch_shapes=[pltpu.VMEM((_B * _R, _NP), jnp.float32)],
    )(planes, gtb, oh_s, oh_o, score)

    sc_sub = pl.kernel(
        _sc_body,
        mesh=plsc.VectorSubcoreMesh(core_axis_name="c", subcore_axis_name="s"),
        out_type=jax.ShapeDtypeStruct((_B, _NP), jnp.float32),
        scratch_types=[
            pltpu.VMEM((_NP,), jnp.float32),
            pltpu.VMEM((_NP,), jnp.int32),
            pltpu.VMEM((_NP,), jnp.int32),
            pltpu.VMEM((_NP,), jnp.int32),
        ],
        compiler_params=pltpu.CompilerParams(needs_layout_passes=False),
    )
    labels = sc_sub(labels_pre, pb1, pb2, ord1, pv1, ord2, pv2)
    return labels, stats


def kernel(roi_pairs, relpn_cls_score, gt_boxes, gt_relation, im_info,
           num_gt_boxes):
    del im_info, num_gt_boxes
    planes = jnp.transpose(roi_pairs[:, :, 1:9], (0, 2, 1))
    planes = jnp.pad(planes, ((0, 0), (0, 0), (0, _NP - _N)),
                     constant_values=_PAD_COORD)
    gtb = jnp.transpose(gt_boxes[:, :, :4], (0, 2, 1))
    gtb = jnp.pad(gtb, ((0, 0), (0, 4), (0, 128 - _G)))
    valid = (gt_relation[:, :, 2] > 0)[:, :, None]
    cols = jnp.arange(128, dtype=gt_relation.dtype)[None, None, :]
    oh_s = ((gt_relation[:, :, 0][:, :, None] == cols) & valid).astype(jnp.float32)
    oh_o = ((gt_relation[:, :, 1][:, :, None] == cols) & valid).astype(jnp.float32)
    score = jnp.pad(relpn_cls_score[:, :, 0], ((0, 0), (0, _NP - _N)))
    labels, stats = _run(planes, gtb, oh_s, oh_o, score,
                         jnp.asarray(_PB1), jnp.asarray(_PB2),
                         jnp.asarray(_ORD1), jnp.asarray(_PV1),
                         jnp.asarray(_ORD2), jnp.asarray(_PV2))
    return (labels[:, :_N], stats[0, 0], stats[1, 0], stats[2, 0])


# speedup vs baseline: 1.1088x; 1.1088x over previous
import jax
import jax.numpy as jnp
import numpy as np
from jax import lax
from jax.experimental import pallas as pl
from jax.experimental.pallas import tpu as pltpu
from jax.experimental.pallas import tpu_sc as plsc

_B, _N, _G, _R = 4, 5000, 50, 64
_NP = 5120
_NEG_OV = 0.3
_POS_OV = 0.7
_NUM_FG = 128
_BATCH = 256
_CHUNK = 512
_PAD_COORD = -1.0e4


def _tf2x32(k1, k2, x1, x2):
    R0 = (13, 15, 26, 6)
    R1 = (17, 29, 16, 24)
    ks = [np.uint32(k1), np.uint32(k2)]
    ks.append(ks[0] ^ ks[1] ^ np.uint32(0x1BD11BDA))
    x = [x1.astype(np.uint32) + ks[0], x2.astype(np.uint32) + ks[1]]

    def rounds(x, rots):
        for r in rots:
            x0 = x[0] + x[1]
            x1r = (x[1] << np.uint32(r)) | (x[1] >> np.uint32(32 - r))
            x = [x0, x0 ^ x1r]
        return x

    for i, (rots, ka, kb) in enumerate(
            [(R0, 1, 2), (R1, 2, 0), (R0, 0, 1), (R1, 1, 2), (R0, 2, 0)]):
        x = rounds(x, rots)
        x = [x[0] + ks[ka], x[1] + ks[kb] + np.uint32(i + 1)]
    return x


def _np_uniform(key, shape):
    n = int(np.prod(shape))
    lo = np.arange(n, dtype=np.uint32)
    hi = np.zeros(n, dtype=np.uint32)
    b1, b2 = _tf2x32(key[0], key[1], hi, lo)
    bits = b1 ^ b2
    fb = (bits >> np.uint32(9)) | np.uint32(0x3F800000)
    return (fb.view(np.float32) - np.float32(1.0)).reshape(shape)


def _np_key42_pris():
    b1, b2 = _tf2x32(np.uint32(0), np.uint32(42),
                     np.zeros(2, np.uint32), np.arange(2, dtype=np.uint32))
    keys = np.stack([b1, b2], axis=1)
    return (_np_uniform(keys[0], (_B, _N)), _np_uniform(keys[1], (_B, _N)))


_PRI1, _PRI2 = _np_key42_pris()


def _pad_bits(pri):
    out = np.full((_B, _NP), -1, np.int32)
    out[:, :_N] = pri.view(np.int32)
    return out


_PB1 = _pad_bits(_PRI1)
_PB2 = _pad_bits(_PRI2)


def _np_order(pri):
    bits = pri.view(np.int32)
    order = np.argsort(-pri, axis=1)
    ordp = np.concatenate(
        [order, np.tile(np.arange(_N, _NP), (_B, 1))], axis=1).astype(np.int32)
    pv = np.take_along_axis(bits, order, axis=1)
    pvp = np.concatenate([pv, np.full((_B, _NP - _N), -1, np.int32)], axis=1)
    return ordp, pvp


_ORD1, _PV1 = _np_order(_PRI1)
_ORD2, _PV2 = _np_order(_PRI2)


def _kth_threshold(mask, bits, k):
    def body(_, lohi):
        lo, hi = lohi
        mid = (lo + hi) // 2
        cnt = jnp.sum((mask & (bits >= mid)).astype(jnp.int32), axis=1,
                      keepdims=True)
        ge = cnt >= k
        return (jnp.where(ge, mid, lo), jnp.where(ge, hi, mid))

    init = (jnp.zeros((_B, 1), jnp.int32),
            jnp.full((_B, 1), 0x40000000, jnp.int32))
    lo, _ = lax.fori_loop(0, 31, body, init)
    return lo


def _body(planes_ref, gtb_ref, oh_s_ref, oh_o_ref, score_ref,
          labels_ref, stats_ref, ov_scr):
    for b in range(_B):
        oh_s = oh_s_ref[b]
        oh_o = oh_o_ref[b]
        gtb = gtb_ref[b]

        def sel(oh, row):
            return jnp.sum(oh * gtb[row:row + 1, :], axis=1, keepdims=True)

        gsx1, gsy1, gsx2, gsy2 = (sel(oh_s, 0), sel(oh_s, 1), sel(oh_s, 2),
                                  sel(oh_s, 3))
        gox1, goy1, gox2, goy2 = (sel(oh_o, 0), sel(oh_o, 1), sel(oh_o, 2),
                                  sel(oh_o, 3))
        vld = jnp.sum(oh_s, axis=1, keepdims=True) > 0.0
        ags = (gsx2 - gsx1 + 1.0) * (gsy2 - gsy1 + 1.0)
        ago = (gox2 - gox1 + 1.0) * (goy2 - goy1 + 1.0)

        for j in range(_NP // _CHUNK):
            ch = planes_ref[b, :, j * _CHUNK:(j + 1) * _CHUNK]
            rsx1, rsy1, rsx2, rsy2 = (ch[0:1], ch[1:2], ch[2:3], ch[3:4])
            rox1, roy1, rox2, roy2 = (ch[4:5], ch[5:6], ch[6:7], ch[7:8])
            ars = (rsx2 - rsx1 + 1.0) * (rsy2 - rsy1 + 1.0)
            aro = (rox2 - rox1 + 1.0) * (roy2 - roy1 + 1.0)

            iw_s = jnp.minimum(rsx2, gsx2) - jnp.maximum(rsx1, gsx1) + 1.0
            ih_s = jnp.minimum(rsy2, gsy2) - jnp.maximum(rsy1, gsy1) + 1.0
            inter_s = jnp.clip(iw_s, 0.0) * jnp.clip(ih_s, 0.0)
            iou_s = inter_s / (ars + ags - inter_s)

            iw_o = jnp.minimum(rox2, gox2) - jnp.maximum(rox1, gox1) + 1.0
            ih_o = jnp.minimum(roy2, goy2) - jnp.maximum(roy1, goy1) + 1.0
            inter_o = jnp.clip(iw_o, 0.0) * jnp.clip(ih_o, 0.0)
            iou_o = inter_o / (aro + ago - inter_o)

            ov_scr[b * _R:(b + 1) * _R, j * _CHUNK:(j + 1) * _CHUNK] = (
                jnp.where(vld, iou_s * iou_o, 0.0))

    ov = ov_scr[:, :]
    gmax = jnp.max(ov, axis=1, keepdims=True)
    gmax_adj = jnp.where(gmax == 0.0, 1e-5, gmax)
    maxov_rows = []
    keep_rows = []
    for b in range(_B):
        ovb = ov[b * _R:(b + 1) * _R]
        maxov_rows.append(jnp.max(ovb, axis=0, keepdims=True))
        keep_rows.append(
            jnp.any(ovb == gmax_adj[b * _R:(b + 1) * _R], axis=0,
                    keepdims=True).astype(jnp.float32))
    maxov = jnp.concatenate(maxov_rows, axis=0)
    keep = jnp.concatenate(keep_rows, axis=0) > 0.0

    labels = jnp.full((_B, _NP), -1.0, jnp.float32)
    labels = jnp.where(maxov < _NEG_OV, 0.0, labels)
    labels = jnp.where(keep, 1.0, labels)
    labels = jnp.where(maxov >= _POS_OV, 1.0, labels)
    idx = lax.broadcasted_iota(jnp.int32, (_B, _NP), 1)
    labels = jnp.where(idx < _N, labels, -1.0)

    score = score_ref[:, :]
    fg = labels == 1.0
    bg = labels == 0.0
    pre = (score > 0.8) & (idx < 300)
    all_fg = jnp.sum(fg.astype(jnp.float32))
    all_bg = jnp.sum(bg.astype(jnp.float32))
    rights_fg = jnp.sum((fg & (score >= 0.5)).astype(jnp.float32))
    rights_bg = jnp.sum((bg & (score < 0.5)).astype(jnp.float32))
    n_pre_correct = jnp.sum((pre & fg).astype(jnp.float32))
    all_fg_pre = jnp.sum(pre.astype(jnp.float32))
    r_fg = rights_fg / jnp.maximum(all_fg, 1.0)
    r_bg = rights_bg / jnp.maximum(all_bg, 1.0)
    acc = n_pre_correct / (1e-5 + all_fg_pre)
    row = lax.broadcasted_iota(jnp.int32, (8, 128), 0)
    stats_ref[...] = jnp.where(
        row == 0, r_fg, jnp.where(row == 1, r_bg, jnp.where(row == 2, acc,
                                                            0.0)))

    labels_ref[...] = labels


_SC_VECS = _NP // 16
_INT_MIN = -2147483647 - 1


def _sc_body(lab_hbm, pb1_hbm, pb2_hbm, ord1_hbm, pv1_hbm, ord2_hbm, pv2_hbm,
             out_hbm, lab_v, pb_v, ord_v, pv_v):
    c = lax.axis_index("c")
    s = lax.axis_index("s")
    b = c * 2 + s

    @pl.when(s < 2)
    def _():
        pltpu.sync_copy(lab_hbm.at[b], lab_v)
        zi = jnp.zeros((16,), jnp.int32)

        def walk(tgt, k):
            def cond(st):
                i, cnt, _ = st
                return (i < _SC_VECS // 2) & (cnt <= k)

            def body(st):
                i, cnt, thr = st
                for u in range(2):
                    sl = pl.ds((i * 2 + u) * 16, 16)
                    lg = plsc.load_gather(lab_v, [ord_v[sl]])
                    mi = (lg == tgt).astype(jnp.int32)
                    cs = plsc.cumsum(mi)
                    tgt_k = jnp.full((16,), k - cnt, jnp.int32)
                    hit = (cs == tgt_k) & (mi > 0)
                    hitv = jnp.max(jnp.where(hit, pv_v[sl], zi))
                    thr = jnp.maximum(thr, hitv)
                    cnt = cnt + jnp.sum(mi)
                return (i + 1, cnt, thr)

            _, cnt, thr = lax.while_loop(
                cond, body, (jnp.int32(0), jnp.int32(0), jnp.int32(0)))
            return thr, cnt > k

        pltpu.sync_copy(ord1_hbm.at[b], ord_v)
        pltpu.sync_copy(pv1_hbm.at[b], pv_v)
        pltpu.sync_copy(pb1_hbm.at[b], pb_v)
        thr1, gate1 = walk(1.0, jnp.int32(_NUM_FG))
        thr1e = jnp.where(gate1, thr1, jnp.int32(_INT_MIN))

        thr1ev = jnp.full((16,), thr1e, jnp.int32)
        thr1v = jnp.full((16,), thr1, jnp.int32)

        def fg_apply(i, acc):
            sel, fgc = acc
            for u in range(4):
                sl = pl.ds((i * 4 + u) * 16, 16)
                lv = lab_v[sl]
                pb = pb_v[sl]
                fgm = lv == 1.0
                lab_v[sl] = jnp.where(fgm & (pb < thr1ev), -1.0, lv)
                sel = sel + (fgm & (pb >= thr1v)).astype(jnp.int32)
                fgc = fgc + fgm.astype(jnp.int32)
            return sel, fgc

        sel, fgc = lax.fori_loop(0, _SC_VECS // 4, fg_apply, (zi, zi))
        fg_sel = jnp.where(gate1, jnp.sum(sel), jnp.sum(fgc))
        num_bg = _BATCH - fg_sel

        pltpu.sync_copy(ord2_hbm.at[b], ord_v)
        pltpu.sync_copy(pv2_hbm.at[b], pv_v)
        pltpu.sync_copy(pb2_hbm.at[b], pb_v)
        thr2, gate2 = walk(0.0, jnp.clip(num_bg, 1, _N))
        thr2e = jnp.where(gate2, thr2, jnp.int32(_INT_MIN))

        thr2ev = jnp.full((16,), thr2e, jnp.int32)

        def bg_apply(i, carry):
            for u in range(4):
                sl = pl.ds((i * 4 + u) * 16, 16)
                lv = lab_v[sl]
                pb = pb_v[sl]
                lab_v[sl] = jnp.where((lv == 0.0) & (pb < thr2ev), -1.0, lv)
            return carry

        lax.fori_loop(0, _SC_VECS // 4, bg_apply, jnp.int32(0))
        pltpu.sync_copy(lab_v, out_hbm.at[b])


@jax.jit
def _run(planes, gtb, oh_s, oh_o, score, pb1, pb2, ord1, pv1, ord2, pv2):
    labels_pre, stats = pl.pallas_call(
        _body,
        out_shape=[
            jax.ShapeDtypeStruct((_B, _NP), jnp.float32),
            jax.ShapeDtypeStruct((8, 128), jnp.float32),
        ],
        scratch_shapes=[pltpu.VMEM((_B * _R, _NP), jnp.float32)],
    )(planes, gtb, oh_s, oh_o, score)

    sc_sub = pl.kernel(
        _sc_body,
        mesh=plsc.VectorSubcoreMesh(core_axis_name="c", subcore_axis_name="s"),
        out_type=jax.ShapeDtypeStruct((_B, _NP), jnp.float32),
        scratch_types=[
            pltpu.VMEM((_NP,), jnp.float32),
            pltpu.VMEM((_NP,), jnp.int32),
            pltpu.VMEM((_NP,), jnp.int32),
            pltpu.VMEM((_NP,), jnp.int32),
        ],
        compiler_params=pltpu.CompilerParams(needs_layout_passes=False),
    )
    labels = sc_sub(labels_pre, pb1, pb2, ord1, pv1, ord2, pv2)
    return labels, stats


def kernel(roi_pairs, relpn_cls_score, gt_boxes, gt_relation, im_info,
           num_gt_boxes):
    del im_info, num_gt_boxes
    planes = jnp.transpose(roi_pairs[:, :, 1:9], (0, 2, 1))
    planes = jnp.pad(planes, ((0, 0), (0, 0), (0, _NP - _N)),
                     constant_values=_PAD_COORD)
    gtb = jnp.transpose(gt_boxes[:, :, :4], (0, 2, 1))
    gtb = jnp.pad(gtb, ((0, 0), (0, 4), (0, 128 - _G)))
    valid = (gt_relation[:, :, 2] > 0)[:, :, None]
    cols = jnp.arange(128, dtype=gt_relation.dtype)[None, None, :]
    oh_s = ((gt_relation[:, :, 0][:, :, None] == cols) & valid).astype(jnp.float32)
    oh_o = ((gt_relation[:, :, 1][:, :, None] == cols) & valid).astype(jnp.float32)
    score = jnp.pad(relpn_cls_score[:, :, 0], ((0, 0), (0, _NP - _N)))
    labels, stats = _run(planes, gtb, oh_s, oh_o, score,
                         jnp.asarray(_PB1), jnp.asarray(_PB2),
                         jnp.asarray(_ORD1), jnp.asarray(_PV1),
                         jnp.asarray(_ORD2), jnp.asarray(_PV2))
    return (labels[:, :_N], stats[0, 0], stats[1, 0], stats[2, 0])


# speedup vs baseline: 1.1303x; 1.0194x over previous
import jax
import jax.numpy as jnp
import numpy as np
from jax import lax
from jax.experimental import pallas as pl
from jax.experimental.pallas import tpu as pltpu
from jax.experimental.pallas import tpu_sc as plsc

_B, _N, _G, _R = 4, 5000, 50, 64
_NP = 5120
_NEG_OV = 0.3
_POS_OV = 0.7
_NUM_FG = 128
_BATCH = 256
_CHUNK = 512
_PAD_COORD = -1.0e4


def _tf2x32(k1, k2, x1, x2):
    R0 = (13, 15, 26, 6)
    R1 = (17, 29, 16, 24)
    ks = [np.uint32(k1), np.uint32(k2)]
    ks.append(ks[0] ^ ks[1] ^ np.uint32(0x1BD11BDA))
    x = [x1.astype(np.uint32) + ks[0], x2.astype(np.uint32) + ks[1]]

    def rounds(x, rots):
        for r in rots:
            x0 = x[0] + x[1]
            x1r = (x[1] << np.uint32(r)) | (x[1] >> np.uint32(32 - r))
            x = [x0, x0 ^ x1r]
        return x

    for i, (rots, ka, kb) in enumerate(
            [(R0, 1, 2), (R1, 2, 0), (R0, 0, 1), (R1, 1, 2), (R0, 2, 0)]):
        x = rounds(x, rots)
        x = [x[0] + ks[ka], x[1] + ks[kb] + np.uint32(i + 1)]
    return x


def _np_uniform(key, shape):
    n = int(np.prod(shape))
    lo = np.arange(n, dtype=np.uint32)
    hi = np.zeros(n, dtype=np.uint32)
    b1, b2 = _tf2x32(key[0], key[1], hi, lo)
    bits = b1 ^ b2
    fb = (bits >> np.uint32(9)) | np.uint32(0x3F800000)
    return (fb.view(np.float32) - np.float32(1.0)).reshape(shape)


def _np_key42_pris():
    b1, b2 = _tf2x32(np.uint32(0), np.uint32(42),
                     np.zeros(2, np.uint32), np.arange(2, dtype=np.uint32))
    keys = np.stack([b1, b2], axis=1)
    return (_np_uniform(keys[0], (_B, _N)), _np_uniform(keys[1], (_B, _N)))


_PRI1, _PRI2 = _np_key42_pris()


def _pad_bits(pri):
    out = np.full((_B, _NP), -1, np.int32)
    out[:, :_N] = pri.view(np.int32)
    return out


_PB1 = _pad_bits(_PRI1)
_PB2 = _pad_bits(_PRI2)


def _np_order(pri):
    bits = pri.view(np.int32)
    order = np.argsort(-pri, axis=1)
    ordp = np.concatenate(
        [order, np.tile(np.arange(_N, _NP), (_B, 1))], axis=1).astype(np.int32)
    pv = np.take_along_axis(bits, order, axis=1)
    pvp = np.concatenate([pv, np.full((_B, _NP - _N), -1, np.int32)], axis=1)
    return ordp, pvp


_ORD1, _PV1 = _np_order(_PRI1)
_ORD2, _PV2 = _np_order(_PRI2)


def _kth_threshold(mask, bits, k):
    def body(_, lohi):
        lo, hi = lohi
        mid = (lo + hi) // 2
        cnt = jnp.sum((mask & (bits >= mid)).astype(jnp.int32), axis=1,
                      keepdims=True)
        ge = cnt >= k
        return (jnp.where(ge, mid, lo), jnp.where(ge, hi, mid))

    init = (jnp.zeros((_B, 1), jnp.int32),
            jnp.full((_B, 1), 0x40000000, jnp.int32))
    lo, _ = lax.fori_loop(0, 31, body, init)
    return lo


def _body(planes_ref, gtb_ref, oh_s_ref, oh_o_ref, score_ref,
          labels_ref, stats_ref, ov_scr):
    for b in range(_B):
        oh_s = oh_s_ref[b]
        oh_o = oh_o_ref[b]
        gtb = gtb_ref[b]

        def sel(oh, row):
            return jnp.sum(oh * gtb[row:row + 1, :], axis=1, keepdims=True)

        gsx1, gsy1, gsx2, gsy2 = (sel(oh_s, 0), sel(oh_s, 1), sel(oh_s, 2),
                                  sel(oh_s, 3))
        gox1, goy1, gox2, goy2 = (sel(oh_o, 0), sel(oh_o, 1), sel(oh_o, 2),
                                  sel(oh_o, 3))
        vld = jnp.sum(oh_s, axis=1, keepdims=True) > 0.0
        ags = (gsx2 - gsx1 + 1.0) * (gsy2 - gsy1 + 1.0)
        ago = (gox2 - gox1 + 1.0) * (goy2 - goy1 + 1.0)

        for j in range(_NP // _CHUNK):
            ch = planes_ref[b, :, j * _CHUNK:(j + 1) * _CHUNK]
            rsx1, rsy1, rsx2, rsy2 = (ch[0:1], ch[1:2], ch[2:3], ch[3:4])
            rox1, roy1, rox2, roy2 = (ch[4:5], ch[5:6], ch[6:7], ch[7:8])
            ars = (rsx2 - rsx1 + 1.0) * (rsy2 - rsy1 + 1.0)
            aro = (rox2 - rox1 + 1.0) * (roy2 - roy1 + 1.0)

            iw_s = jnp.minimum(rsx2, gsx2) - jnp.maximum(rsx1, gsx1) + 1.0
            ih_s = jnp.minimum(rsy2, gsy2) - jnp.maximum(rsy1, gsy1) + 1.0
            inter_s = jnp.clip(iw_s, 0.0) * jnp.clip(ih_s, 0.0)
            iou_s = inter_s / (ars + ags - inter_s)

            iw_o = jnp.minimum(rox2, gox2) - jnp.maximum(rox1, gox1) + 1.0
            ih_o = jnp.minimum(roy2, goy2) - jnp.maximum(roy1, goy1) + 1.0
            inter_o = jnp.clip(iw_o, 0.0) * jnp.clip(ih_o, 0.0)
            iou_o = inter_o / (aro + ago - inter_o)

            ov_scr[b * _R:(b + 1) * _R, j * _CHUNK:(j + 1) * _CHUNK] = (
                jnp.where(vld, iou_s * iou_o, 0.0))

    ov = ov_scr[:, :]
    gmax = jnp.max(ov, axis=1, keepdims=True)
    gmax_adj = jnp.where(gmax == 0.0, 1e-5, gmax)
    maxov_rows = []
    keep_rows = []
    for b in range(_B):
        ovb = ov[b * _R:(b + 1) * _R]
        maxov_rows.append(jnp.max(ovb, axis=0, keepdims=True))
        keep_rows.append(
            jnp.any(ovb == gmax_adj[b * _R:(b + 1) * _R], axis=0,
                    keepdims=True).astype(jnp.float32))
    maxov = jnp.concatenate(maxov_rows, axis=0)
    keep = jnp.concatenate(keep_rows, axis=0) > 0.0

    labels = jnp.full((_B, _NP), -1.0, jnp.float32)
    labels = jnp.where(maxov < _NEG_OV, 0.0, labels)
    labels = jnp.where(keep, 1.0, labels)
    labels = jnp.where(maxov >= _POS_OV, 1.0, labels)
    idx = lax.broadcasted_iota(jnp.int32, (_B, _NP), 1)
    labels = jnp.where(idx < _N, labels, -1.0)

    score = score_ref[:, :]
    fg = labels == 1.0
    bg = labels == 0.0
    pre = (score > 0.8) & (idx < 300)
    all_fg = jnp.sum(fg.astype(jnp.float32))
    all_bg = jnp.sum(bg.astype(jnp.float32))
    rights_fg = jnp.sum((fg & (score >= 0.5)).astype(jnp.float32))
    rights_bg = jnp.sum((bg & (score < 0.5)).astype(jnp.float32))
    n_pre_correct = jnp.sum((pre & fg).astype(jnp.float32))
    all_fg_pre = jnp.sum(pre.astype(jnp.float32))
    r_fg = rights_fg / jnp.maximum(all_fg, 1.0)
    r_bg = rights_bg / jnp.maximum(all_bg, 1.0)
    acc = n_pre_correct / (1e-5 + all_fg_pre)
    row = lax.broadcasted_iota(jnp.int32, (8, 128), 0)
    stats_ref[...] = jnp.where(
        row == 0, r_fg, jnp.where(row == 1, r_bg, jnp.where(row == 2, acc,
                                                            0.0)))

    labels_ref[...] = labels


_SC_VECS = _NP // 16
_INT_MIN = -2147483647 - 1


def _sc_body(lab_hbm, pb1_hbm, pb2_hbm, ord1_hbm, pv1_hbm, ord2_hbm, pv2_hbm,
             out_hbm, lab_v, pb_v, ord_v, pv_v):
    c = lax.axis_index("c")
    s = lax.axis_index("s")
    b = c * 2 + s

    @pl.when(s < 2)
    def _():
        pltpu.sync_copy(lab_hbm.at[b], lab_v)
        zi = jnp.zeros((16,), jnp.int32)

        def walk(tgt, k):
            def cond(st):
                i, cnt, _ = st
                return (i < _SC_VECS // 4) & (cnt <= k)

            def body(st):
                i, cnt, thr = st
                for u in range(4):
                    sl = pl.ds((i * 4 + u) * 16, 16)
                    lg = plsc.load_gather(lab_v, [ord_v[sl]])
                    mi = (lg == tgt).astype(jnp.int32)
                    cs = plsc.cumsum(mi)
                    tgt_k = jnp.full((16,), k - cnt, jnp.int32)
                    hit = (cs == tgt_k) & (mi > 0)
                    hitv = jnp.max(jnp.where(hit, pv_v[sl], zi))
                    thr = jnp.maximum(thr, hitv)
                    cnt = cnt + jnp.sum(mi)
                return (i + 1, cnt, thr)

            _, cnt, thr = lax.while_loop(
                cond, body, (jnp.int32(0), jnp.int32(0), jnp.int32(0)))
            return thr, cnt > k

        pltpu.sync_copy(ord1_hbm.at[b], ord_v)
        pltpu.sync_copy(pv1_hbm.at[b], pv_v)
        pltpu.sync_copy(pb1_hbm.at[b], pb_v)
        thr1, gate1 = walk(1.0, jnp.int32(_NUM_FG))
        thr1e = jnp.where(gate1, thr1, jnp.int32(_INT_MIN))

        thr1ev = jnp.full((16,), thr1e, jnp.int32)
        thr1v = jnp.full((16,), thr1, jnp.int32)

        def fg_apply(i, acc):
            sel, fgc = acc
            for u in range(4):
                sl = pl.ds((i * 4 + u) * 16, 16)
                lv = lab_v[sl]
                pb = pb_v[sl]
                fgm = lv == 1.0
                lab_v[sl] = jnp.where(fgm & (pb < thr1ev), -1.0, lv)
                sel = sel + (fgm & (pb >= thr1v)).astype(jnp.int32)
                fgc = fgc + fgm.astype(jnp.int32)
            return sel, fgc

        sel, fgc = lax.fori_loop(0, _SC_VECS // 4, fg_apply, (zi, zi))
        fg_sel = jnp.where(gate1, jnp.sum(sel), jnp.sum(fgc))
        num_bg = _BATCH - fg_sel

        pltpu.sync_copy(ord2_hbm.at[b], ord_v)
        pltpu.sync_copy(pv2_hbm.at[b], pv_v)
        pltpu.sync_copy(pb2_hbm.at[b], pb_v)
        thr2, gate2 = walk(0.0, jnp.clip(num_bg, 1, _N))
        thr2e = jnp.where(gate2, thr2, jnp.int32(_INT_MIN))

        thr2ev = jnp.full((16,), thr2e, jnp.int32)

        def bg_apply(i, carry):
            for u in range(4):
                sl = pl.ds((i * 4 + u) * 16, 16)
                lv = lab_v[sl]
                pb = pb_v[sl]
                lab_v[sl] = jnp.where((lv == 0.0) & (pb < thr2ev), -1.0, lv)
            return carry

        lax.fori_loop(0, _SC_VECS // 4, bg_apply, jnp.int32(0))
        pltpu.sync_copy(lab_v, out_hbm.at[b])


@jax.jit
def _run(planes, gtb, oh_s, oh_o, score, pb1, pb2, ord1, pv1, ord2, pv2):
    labels_pre, stats = pl.pallas_call(
        _body,
        out_shape=[
            jax.ShapeDtypeStruct((_B, _NP), jnp.float32),
            jax.ShapeDtypeStruct((8, 128), jnp.float32),
        ],
        scratch_shapes=[pltpu.VMEM((_B * _R, _NP), jnp.float32)],
    )(planes, gtb, oh_s, oh_o, score)

    sc_sub = pl.kernel(
        _sc_body,
        mesh=plsc.VectorSubcoreMesh(core_axis_name="c", subcore_axis_name="s"),
        out_type=jax.ShapeDtypeStruct((_B, _NP), jnp.float32),
        scratch_types=[
            pltpu.VMEM((_NP,), jnp.float32),
            pltpu.VMEM((_NP,), jnp.int32),
            pltpu.VMEM((_NP,), jnp.int32),
            pltpu.VMEM((_NP,), jnp.int32),
        ],
        compiler_params=pltpu.CompilerParams(needs_layout_passes=False),
    )
    labels = sc_sub(labels_pre, pb1, pb2, ord1, pv1, ord2, pv2)
    return labels, stats


def kernel(roi_pairs, relpn_cls_score, gt_boxes, gt_relation, im_info,
           num_gt_boxes):
    del im_info, num_gt_boxes
    planes = jnp.transpose(roi_pairs[:, :, 1:9], (0, 2, 1))
    planes = jnp.pad(planes, ((0, 0), (0, 0), (0, _NP - _N)),
                     constant_values=_PAD_COORD)
    gtb = jnp.transpose(gt_boxes[:, :, :4], (0, 2, 1))
    gtb = jnp.pad(gtb, ((0, 0), (0, 4), (0, 128 - _G)))
    valid = (gt_relation[:, :, 2] > 0)[:, :, None]
    cols = jnp.arange(128, dtype=gt_relation.dtype)[None, None, :]
    oh_s = ((gt_relation[:, :, 0][:, :, None] == cols) & valid).astype(jnp.float32)
    oh_o = ((gt_relation[:, :, 1][:, :, None] == cols) & valid).astype(jnp.float32)
    score = jnp.pad(relpn_cls_score[:, :, 0], ((0, 0), (0, _NP - _N)))
    labels, stats = _run(planes, gtb, oh_s, oh_o, score,
                         jnp.asarray(_PB1), jnp.asarray(_PB2),
                         jnp.asarray(_ORD1), jnp.asarray(_PV1),
                         jnp.asarray(_ORD2), jnp.asarray(_PV2))
    return (labels[:, :_N], stats[0, 0], stats[1, 0], stats[2, 0])
